# baseline (speedup 1.0000x reference)
import functools

import jax
import jax.numpy as jnp
from jax import lax
from jax.experimental import pallas as pl
from jax.experimental.pallas import tpu as pltpu

_D = 64
_B = 16384
_STEP = 256
_STEPS = _B // _STEP


def _tc_body(idx_ref, table_ref, out_block, buf, sem):
    i = pl.program_id(0)
    for j in range(_STEP):
        r = idx_ref[i * _STEP + j]
        pltpu.make_async_copy(
            table_ref.at[pl.ds(r, 1)], buf.at[pl.ds(j, 1)], sem
        ).start()
    pltpu.make_async_copy(table_ref.at[pl.ds(0, _STEP)], buf, sem).wait()
    out_block[...] = buf[...]


_tc_gather = pl.pallas_call(
    _tc_body,
    grid_spec=pltpu.PrefetchScalarGridSpec(
        num_scalar_prefetch=1,
        grid=(_STEPS,),
        in_specs=[pl.BlockSpec(memory_space=pl.ANY)],
        out_specs=pl.BlockSpec((_STEP, _D), lambda i, idx: (i, 0)),
        scratch_shapes=[
            pltpu.VMEM((_STEP, _D), jnp.float32),
            pltpu.SemaphoreType.DMA,
        ],
    ),
    out_shape=jax.ShapeDtypeStruct((_B, _D), jnp.float32),
)


@jax.jit
def kernel(ori, dest, table):
    return _tc_gather(ori, table), _tc_gather(dest, table)


# speedup vs baseline: 1.2319x; 1.2319x over previous
import functools

import jax
import jax.numpy as jnp
from jax import lax
from jax.experimental import pallas as pl
from jax.experimental.pallas import tpu as pltpu
from jax.experimental.pallas import tpu_sc as plsc

_D = 64
_B = 16384
_NC = 2
_NS = 16
_NW = _NC * _NS

_B_SC = 10240
_B_TC = _B - _B_SC
_RW = _B_SC // _NW
_GROUPS = _RW // 16

_mesh = plsc.VectorSubcoreMesh(core_axis_name="c", subcore_axis_name="s")


@functools.partial(
    pl.kernel,
    out_type=(
        jax.ShapeDtypeStruct((_B_SC, _D), jnp.float32),
        jax.ShapeDtypeStruct((_B_SC, _D), jnp.float32),
    ),
    mesh=_mesh,
    scratch_types=[
        pltpu.VMEM((_RW,), jnp.int32),
        pltpu.VMEM((_RW,), jnp.int32),
        pltpu.VMEM((_RW, _D), jnp.float32),
        pltpu.VMEM((_RW, _D), jnp.float32),
        pltpu.SemaphoreType.DMA,
        pltpu.SemaphoreType.DMA,
    ],
)
def _sc_gather(table, ori, dest, o_out, d_out, oidx_v, didx_v, obuf, dbuf,
               sem_o, sem_d):
    wid = lax.axis_index("s") * _NC + lax.axis_index("c")
    row0 = wid * _RW
    pltpu.sync_copy(ori.at[pl.ds(row0, _RW)], oidx_v)
    pltpu.sync_copy(dest.at[pl.ds(row0, _RW)], didx_v)

    def group_body(g, _):
        ovec = oidx_v[pl.ds(g * 16, 16)]
        dvec = didx_v[pl.ds(g * 16, 16)]
        for l in range(16):
            pltpu.async_copy(
                table.at[pl.ds(ovec[l], 1)], obuf.at[pl.ds(g * 16 + l, 1)], sem_o
            )
            pltpu.async_copy(
                table.at[pl.ds(dvec[l], 1)], dbuf.at[pl.ds(g * 16 + l, 1)], sem_d
            )
        return ()

    lax.fori_loop(0, _GROUPS, group_body, ())
    pltpu.make_async_copy(table.at[pl.ds(0, _RW)], obuf, sem_o).wait()
    pltpu.make_async_copy(table.at[pl.ds(0, _RW)], dbuf, sem_d).wait()
    pltpu.sync_copy(obuf, o_out.at[pl.ds(row0, _RW)])
    pltpu.sync_copy(dbuf, d_out.at[pl.ds(row0, _RW)])


_STEP = 256
_STEPS = _B_TC // _STEP


def _tc_body(idx_ref, table_ref, out_block, buf, sem):
    i = pl.program_id(0)
    for j in range(_STEP):
        r = idx_ref[i * _STEP + j]
        pltpu.make_async_copy(
            table_ref.at[pl.ds(r, 1)], buf.at[pl.ds(j, 1)], sem
        ).start()
    pltpu.make_async_copy(table_ref.at[pl.ds(0, _STEP)], buf, sem).wait()
    out_block[...] = buf[...]


_tc_gather = pl.pallas_call(
    _tc_body,
    grid_spec=pltpu.PrefetchScalarGridSpec(
        num_scalar_prefetch=1,
        grid=(_STEPS,),
        in_specs=[pl.BlockSpec(memory_space=pl.ANY)],
        out_specs=pl.BlockSpec((_STEP, _D), lambda i, idx: (i, 0)),
        scratch_shapes=[
            pltpu.VMEM((_STEP, _D), jnp.float32),
            pltpu.SemaphoreType.DMA,
        ],
    ),
    out_shape=jax.ShapeDtypeStruct((_B_TC, _D), jnp.float32),
)


@jax.jit
def kernel(ori, dest, table):
    o_sc, d_sc = _sc_gather(table, ori[:_B_SC], dest[:_B_SC])
    o_tc = _tc_gather(ori[_B_SC:], table)
    d_tc = _tc_gather(dest[_B_SC:], table)
    emb_o = jnp.concatenate([o_sc, o_tc], axis=0)
    emb_d = jnp.concatenate([d_sc, d_tc], axis=0)
    return emb_o, emb_d
